# Initial kernel scaffold; baseline (speedup 1.0000x reference)
import jax
import jax.numpy as jnp
from jax.experimental import pallas as pl


def kernel(x, table):
    raise NotImplementedError("write your pallas kernel here")


# speedup vs baseline: 1.1032x; 1.1032x over previous
import functools

import jax
import jax.numpy as jnp
from jax import lax
from jax.experimental import pallas as pl
from jax.experimental.pallas import tpu as pltpu
from jax.experimental.pallas import tpu_sc as plsc

DIM = 32
SUB = 128
GROUP = 8


@functools.partial(jax.jit, static_argnames=("b_total",))
def _lookup(x2d, table, b_total):
    info = plsc.get_sparse_core_info()
    nc, ns = info.num_cores, info.num_subcores
    nw = nc * ns
    b_per_w = b_total // nw
    n_sub = b_per_w // SUB
    n_grp = b_per_w // (SUB * GROUP)
    mesh = plsc.VectorSubcoreMesh(core_axis_name="c", subcore_axis_name="s")

    @functools.partial(
        pl.kernel,
        out_type=jax.ShapeDtypeStruct((b_total, DIM), jnp.float32),
        mesh=mesh,
        scratch_types=[
            pltpu.VMEM((n_sub, SUB), jnp.int32),
            pltpu.VMEM((SUB * GROUP, DIM), jnp.float32),
            pltpu.SemaphoreType.DMA,
        ],
        compiler_params=pltpu.CompilerParams(use_tc_tiling_on_sc=False),
    )
    def lookup(x_hbm, table_hbm, out_hbm, idx_v, rows_v, gsem):
        wid = lax.axis_index("s") * nc + lax.axis_index("c")
        pltpu.sync_copy(x_hbm.at[pl.ds(wid * n_sub, n_sub)], idx_v)

        def group_body(g, carry):
            descs = [
                pltpu.async_copy(
                    table_hbm.at[idx_v.at[g * GROUP + j]],
                    rows_v.at[pl.ds(j * SUB, SUB)],
                    gsem,
                )
                for j in range(GROUP)
            ]
            for d in descs:
                d.wait()
            pltpu.sync_copy(
                rows_v,
                out_hbm.at[pl.ds(wid * b_per_w + g * SUB * GROUP, SUB * GROUP)],
            )
            return carry

        lax.fori_loop(0, n_grp, group_body, 0)

    return lookup(x2d, table)


def kernel(x, table):
    b, s = x.shape
    b_total = b * s
    x2d = x.reshape(b_total // SUB, SUB).astype(jnp.int32)
    out = _lookup(x2d, table, b_total)
    return out.reshape(b, s, DIM)


# speedup vs baseline: 1.1131x; 1.0090x over previous
import functools

import jax
import jax.numpy as jnp
from jax import lax
from jax.experimental import pallas as pl
from jax.experimental.pallas import tpu as pltpu
from jax.experimental.pallas import tpu_sc as plsc

DIM = 32
SUB = 128
GROUP = 4
NBUF = 5
LOOKAHEAD = 3
GSZ = SUB * GROUP


@functools.partial(jax.jit, static_argnames=("b_total",))
def _lookup(x2d, table, b_total):
    info = plsc.get_sparse_core_info()
    nc, ns = info.num_cores, info.num_subcores
    nw = nc * ns
    b_per_w = b_total // nw
    n_sub = b_per_w // SUB
    n_grp = b_per_w // GSZ
    mesh = plsc.VectorSubcoreMesh(core_axis_name="c", subcore_axis_name="s")

    @functools.partial(
        pl.kernel,
        out_type=jax.ShapeDtypeStruct((b_total, DIM), jnp.float32),
        mesh=mesh,
        scratch_types=[
            pltpu.VMEM((n_sub, SUB), jnp.int32),
            pltpu.VMEM((NBUF, GSZ, DIM), jnp.float32),
            pltpu.SemaphoreType.DMA((NBUF,)),
            pltpu.SemaphoreType.DMA((NBUF,)),
        ],
        compiler_params=pltpu.CompilerParams(use_tc_tiling_on_sc=False),
    )
    def lookup(x_hbm, table_hbm, out_hbm, idx_v, rows_v, gsem, osem):
        wid = lax.axis_index("s") * nc + lax.axis_index("c")
        out_base = wid * b_per_w
        pltpu.sync_copy(x_hbm.at[pl.ds(wid * n_sub, n_sub)], idx_v)

        def fire(g):
            s = lax.rem(g, NBUF)
            for j in range(GROUP):
                pltpu.async_copy(
                    table_hbm.at[idx_v.at[g * GROUP + j]],
                    rows_v.at[s].at[pl.ds(j * SUB, SUB)],
                    gsem.at[s],
                )

        def drain(g):
            s = lax.rem(g, NBUF)
            for j in range(GROUP):
                pltpu.make_async_copy(
                    table_hbm.at[idx_v.at[g * GROUP + j]],
                    rows_v.at[s].at[pl.ds(j * SUB, SUB)],
                    gsem.at[s],
                ).wait()

        def write(g):
            s = lax.rem(g, NBUF)
            pltpu.async_copy(
                rows_v.at[s],
                out_hbm.at[pl.ds(out_base + g * GSZ, GSZ)],
                osem.at[s],
            )

        def wait_write(g):
            g = lax.max(g, 0)
            s = lax.rem(g, NBUF)
            pltpu.make_async_copy(
                rows_v.at[s],
                out_hbm.at[pl.ds(out_base + g * GSZ, GSZ)],
                osem.at[s],
            ).wait()

        for g in range(LOOKAHEAD):
            fire(g)

        def group_body(g, carry):
            @pl.when(g >= NBUF - LOOKAHEAD)
            def _():
                wait_write(g - (NBUF - LOOKAHEAD))

            @pl.when(g + LOOKAHEAD < n_grp)
            def _():
                fire(g + LOOKAHEAD)

            drain(g)
            write(g)
            return carry

        lax.fori_loop(0, n_grp, group_body, 0)

        for g in range(n_grp - (NBUF - LOOKAHEAD), n_grp):
            wait_write(g)

    return lookup(x2d, table)


def kernel(x, table):
    b, s = x.shape
    b_total = b * s
    x2d = x.reshape(b_total // SUB, SUB).astype(jnp.int32)
    out = _lookup(x2d, table, b_total)
    return out.reshape(b, s, DIM)


# speedup vs baseline: 1.6181x; 1.4537x over previous
import functools

import jax
import jax.numpy as jnp
from jax import lax
from jax.experimental import pallas as pl
from jax.experimental.pallas import tpu as pltpu
from jax.experimental.pallas import tpu_sc as plsc

DIM = 32
SUB = 128


N_FULL_COLS = 7812
REM_ROWS = 64
LA_A = 3
NV_A = 4
NR_A = 2


def _make_retile(nc, ns, mesh):
    nw = nc * ns
    base_cols = N_FULL_COLS // nw
    extra = N_FULL_COLS - base_cols * nw

    @functools.partial(
        pl.kernel,
        out_type=(
            jax.ShapeDtypeStruct((32000000,), jnp.float32),
            jax.ShapeDtypeStruct((128 * 56, SUB), jnp.int32),
        ),
        mesh=mesh,
        scratch_types=[
            pltpu.VMEM((NV_A, DIM, SUB), jnp.float32),
            pltpu.VMEM((NR_A * SUB * DIM,), jnp.float32),
            pltpu.VMEM((REM_ROWS * DIM,), jnp.float32),
            pltpu.VMEM((56, SUB), jnp.int32),
            pltpu.SemaphoreType.DMA((NV_A,)),
            pltpu.SemaphoreType.DMA((NR_A,)),
        ],
        compiler_params=pltpu.CompilerParams(use_tc_tiling_on_sc=True, needs_layout_passes=False),
    )
    def retile(tT_hbm, xT_hbm, tail_hbm, scr_hbm, xp_hbm, vbuf, rbuf, vrem, xbuf, rsem, wsem):
        wid = lax.axis_index("s") * nc + lax.axis_index("c")
        n_my = base_cols + jnp.where(wid < extra, 1, 0)
        colbase = wid * base_cols + lax.min(wid, extra)
        iota16 = lax.iota(jnp.int32, 16)
        iota32 = iota16 * DIM

        def fire_read(t):
            vs = lax.rem(t, NV_A)
            col = colbase + t
            pltpu.async_copy(
                tT_hbm.at[:, pl.ds(col * SUB, SUB)], vbuf.at[vs], rsem.at[vs]
            )

        def drain_read(t):
            vs = lax.rem(t, NV_A)
            col = colbase + t
            pltpu.make_async_copy(
                tT_hbm.at[:, pl.ds(col * SUB, SUB)], vbuf.at[vs], rsem.at[vs]
            ).wait()

        def transpose_block(t):
            vs = lax.rem(t, NV_A)
            roff = lax.rem(t, NR_A) * (SUB * DIM)
            vb = vbuf.at[vs]
            for c in range(DIM):
                for r0 in range(0, SUB, 16):
                    vals = vb[c, pl.ds(r0, 16)]
                    plsc.store_scatter(rbuf, [iota32 + (r0 * DIM + c) + roff], vals)

        def fire_write(t):
            rs = lax.rem(t, NR_A)
            col = colbase + t
            pltpu.async_copy(
                rbuf.at[pl.ds(rs * SUB * DIM, SUB * DIM)],
                scr_hbm.at[pl.ds(col * SUB * DIM, SUB * DIM)],
                wsem.at[rs],
            )

        def wait_write(t):
            t = lax.max(t, 0)
            rs = lax.rem(t, NR_A)
            col = colbase + t
            pltpu.make_async_copy(
                rbuf.at[pl.ds(rs * SUB * DIM, SUB * DIM)],
                scr_hbm.at[pl.ds(col * SUB * DIM, SUB * DIM)],
                wsem.at[rs],
            ).wait()

        for t in range(LA_A):
            fire_read(t)

        def body(t, carry):
            @pl.when(t >= NR_A)
            def _():
                wait_write(t - NR_A)

            @pl.when(t + LA_A < n_my)
            def _():
                fire_read(t + LA_A)

            drain_read(t)
            transpose_block(t)
            fire_write(t)
            return carry

        lax.fori_loop(0, n_my, body, 0)
        wait_write(n_my - 2)
        wait_write(n_my - 1)

        for q in range(4):
            jb = wid * 4 + q
            pltpu.sync_copy(xT_hbm.at[:, pl.ds(jb * SUB, SUB)], xbuf.at[pl.ds(0, 50)])
            pltpu.sync_copy(xbuf, xp_hbm.at[pl.ds(jb * 56, 56)])

        @pl.when(wid == 0)
        def _():
            pltpu.sync_copy(tail_hbm, vrem)
            pltpu.sync_copy(
                vrem,
                scr_hbm.at[pl.ds(N_FULL_COLS * SUB * DIM, REM_ROWS * DIM)],
            )

    return retile


LA_B = 3
NG_B = 4
NT_B = 2
BLOCKS = 6400


def _make_gather(nc, ns, mesh):
    nw = nc * ns
    n_my = BLOCKS // nw

    @functools.partial(
        pl.kernel,
        out_type=jax.ShapeDtypeStruct((26214400,), jnp.float32),
        mesh=mesh,
        scratch_types=[
            pltpu.VMEM((4 * 56, SUB), jnp.int32),
            pltpu.VMEM((NG_B, SUB, DIM), jnp.float32),
            pltpu.VMEM((NT_B * DIM * SUB,), jnp.float32),
            pltpu.SemaphoreType.DMA((NG_B,)),
            pltpu.SemaphoreType.DMA((NT_B,)),
        ],
        compiler_params=pltpu.CompilerParams(use_tc_tiling_on_sc=False, needs_layout_passes=False),
    )
    def gather(scr_hbm, xp_hbm, out_hbm, idx_v, gbuf, tbuf, gsem, wsem):
        wid = lax.axis_index("s") * nc + lax.axis_index("c")
        pbase = wid * n_my
        pltpu.sync_copy(xp_hbm.at[pl.ds(wid * 4 * 56, 4 * 56)], idx_v)
        iota16 = lax.iota(jnp.int32, 16)
        iota128 = iota16 * SUB

        def idx_row(t):
            return (t // 50) * 56 + lax.rem(t, 50)

        def fire_gather(t):
            gs = lax.rem(t, NG_B)
            pltpu.async_copy(
                scr_hbm.at[idx_v.at[idx_row(t)]], gbuf.at[gs], gsem.at[gs]
            )

        def drain_gather(t):
            gs = lax.rem(t, NG_B)
            pltpu.make_async_copy(
                scr_hbm.at[idx_v.at[idx_row(t)]], gbuf.at[gs], gsem.at[gs]
            ).wait()

        def transpose_block(t):
            gs = lax.rem(t, NG_B)
            toff = lax.rem(t, NT_B) * (DIM * SUB)
            g = gbuf.at[gs]
            for i in range(SUB):
                for c0 in range(0, DIM, 16):
                    vals = g[i, pl.ds(c0, 16)]
                    plsc.store_scatter(tbuf, [iota128 + (c0 * SUB + i) + toff], vals)

        def out_off(t, k):
            p = pbase + t
            s = lax.rem(p, 50)
            jb = p // 50
            return ((s * 4 + k) * SUB + jb) * (8 * SUB)

        def fire_writes(t):
            ts = lax.rem(t, NT_B)
            for k in range(4):
                pltpu.async_copy(
                    tbuf.at[pl.ds(ts * DIM * SUB + k * 8 * SUB, 8 * SUB)],
                    out_hbm.at[pl.ds(out_off(t, k), 8 * SUB)],
                    wsem.at[ts],
                )

        def wait_writes(t):
            t = lax.max(t, 0)
            ts = lax.rem(t, NT_B)
            for k in range(4):
                pltpu.make_async_copy(
                    tbuf.at[pl.ds(ts * DIM * SUB + k * 8 * SUB, 8 * SUB)],
                    out_hbm.at[pl.ds(out_off(t, k), 8 * SUB)],
                    wsem.at[ts],
                ).wait()

        for t in range(LA_B):
            fire_gather(t)

        def body(t, carry):
            @pl.when(t >= NT_B)
            def _():
                wait_writes(t - NT_B)

            @pl.when(t + LA_B < n_my)
            def _():
                fire_gather(t + LA_B)

            drain_gather(t)
            transpose_block(t)
            fire_writes(t)
            return carry

        lax.fori_loop(0, n_my, body, 0)
        wait_writes(n_my - 2)
        wait_writes(n_my - 1)

    return gather


@jax.jit
def _lookup(x, table):
    info = plsc.get_sparse_core_info()
    nc, ns = info.num_cores, info.num_subcores
    mesh = plsc.VectorSubcoreMesh(core_axis_name="c", subcore_axis_name="s")
    tail = table[N_FULL_COLS * SUB:, :].reshape(REM_ROWS * DIM)
    scr, xp = _make_retile(nc, ns, mesh)(table.T, x.T.astype(jnp.int32), tail)
    out5 = _make_gather(nc, ns, mesh)(scr.reshape(1000000, DIM), xp)
    return (
        out5.reshape(50, 4, SUB, 8, SUB)
        .transpose(2, 4, 0, 1, 3)
        .reshape(16384, 50, DIM)
    )


def kernel(x, table):
    return _lookup(x, table)
